# baseline (speedup 1.0000x reference)
import jax
import jax.numpy as jnp
from jax.experimental import pallas as pl
from jax.experimental.pallas import tpu as pltpu

_N = 4096
_D_IN = 256
_D_OUT = 256
_TM = 512


def _graph_conv_kernel(a_ref, x_ref, wt_ref, b_ref, o_ref):
    y = jnp.dot(
        x_ref[...].astype(jnp.bfloat16),
        wt_ref[...].astype(jnp.bfloat16),
        preferred_element_type=jnp.float32,
    ).astype(jnp.bfloat16)
    acc = jnp.dot(
        a_ref[...].astype(jnp.bfloat16),
        y,
        preferred_element_type=jnp.float32,
    )
    o_ref[...] = acc + b_ref[...]


def kernel(A, x, W, b):
    wt = W.T
    b2 = b.reshape(1, _D_OUT)
    return pl.pallas_call(
        _graph_conv_kernel,
        grid=(_N // _TM,),
        in_specs=[
            pl.BlockSpec((_TM, _N), lambda i: (i, 0)),
            pl.BlockSpec((_N, _D_IN), lambda i: (0, 0)),
            pl.BlockSpec((_D_IN, _D_OUT), lambda i: (0, 0)),
            pl.BlockSpec((1, _D_OUT), lambda i: (0, 0)),
        ],
        out_specs=pl.BlockSpec((_TM, _D_OUT), lambda i: (i, 0)),
        out_shape=jax.ShapeDtypeStruct((_N, _D_OUT), jnp.float32),
    )(A, x, wt, b2)
